# baseline (speedup 1.0000x reference)
import dataclasses
import functools

import jax
import jax.numpy as jnp
from jax import lax
from jax.experimental import pallas as pl
from jax.experimental.pallas import tpu as pltpu
from jax.experimental.pallas import tpu_sc as plsc

N = 10000
E = 320000
D = 128
NP = 10240
HALF = NP // 2
RPT = HALF // 16
CHUNK = 128
NCHUNKS = E // CHUNK
RB = 1024
ZROW = N

_mesh = plsc.VectorSubcoreMesh(core_axis_name="c", subcore_axis_name="s")

_sc_params = pltpu.CompilerParams()
if "needs_layout_passes" in pltpu.CompilerParams.__dataclass_fields__:
    _sc_params = dataclasses.replace(_sc_params, needs_layout_passes=False)


@functools.partial(
    pl.kernel,
    mesh=_mesh,
    compiler_params=_sc_params,
    out_type=[
        jax.ShapeDtypeStruct((2, HALF, D), jnp.float32),
        jax.ShapeDtypeStruct((2, 16, HALF), jnp.float32),
    ],
    scratch_types=[
        pltpu.VMEM_SHARED((HALF, D), jnp.float32),
        pltpu.VMEM((1, CHUNK), jnp.int32),
        pltpu.VMEM((1, CHUNK), jnp.int32),
        pltpu.VMEM((1, CHUNK), jnp.int32),
        pltpu.VMEM((1, CHUNK), jnp.int32),
        pltpu.VMEM((CHUNK, D), jnp.float32),
        pltpu.VMEM((CHUNK, D), jnp.float32),
        pltpu.VMEM((HALF,), jnp.float32),
        pltpu.SemaphoreType.DMA,
    ],
)
def _sc_segment_sum(xa_hbm, src_hbm, dst_hbm, sum_hbm, cnt_hbm,
                    acc_sh, src_v, dst_v, src2_v, idx2_v, rows_v, zero_v,
                    cnt_v, sem):
    c = lax.axis_index("c")
    s = lax.axis_index("s")
    zeros16 = jnp.zeros((16,), jnp.float32)
    ones16 = jnp.ones((16,), jnp.float32)

    @pl.loop(0, CHUNK)
    def _zrow(i):
        @pl.loop(0, D, step=16)
        def _zcol(j):
            zero_v[i, pl.ds(j, 16)] = zeros16

    @pl.loop(0, HALF, step=16)
    def _zcnt(j):
        cnt_v[pl.ds(j, 16)] = zeros16

    row0 = s * RPT

    @pl.loop(0, 256, step=CHUNK)
    def _zacc(r):
        pltpu.sync_copy(zero_v, acc_sh.at[pl.ds(row0 + r, CHUNK)])

    pltpu.sync_copy(zero_v.at[pl.ds(0, 64)], acc_sh.at[pl.ds(row0 + 256, 64)])

    plsc.subcore_barrier()

    base = c * HALF

    @pl.loop(s, NCHUNKS, step=16)
    def _edges(ch):
        pltpu.sync_copy(src_hbm.at[pl.ds(ch, 1)], src_v)
        pltpu.sync_copy(dst_hbm.at[pl.ds(ch, 1)], dst_v)

        @pl.loop(0, CHUNK, step=16)
        def _route(k):
            d = dst_v[0, pl.ds(k, 16)]
            sv = src_v[0, pl.ds(k, 16)]
            local = d - base
            in_range = (local >= 0) & (local < HALF)
            src2_v[0, pl.ds(k, 16)] = jnp.where(in_range, sv, ZROW)
            loc = jnp.where(in_range, local, 0)
            idx2_v[0, pl.ds(k, 16)] = loc
            plsc.addupdate_scatter(cnt_v, [loc], ones16, mask=in_range)

        pltpu.async_copy(xa_hbm.at[src2_v.at[0]], rows_v, sem).wait()
        pltpu.sync_copy(rows_v, acc_sh.at[idx2_v.at[0]], add=True)

    plsc.subcore_barrier()
    pltpu.sync_copy(acc_sh.at[pl.ds(row0, RPT)],
                    sum_hbm.at[c, pl.ds(row0, RPT)])
    pltpu.sync_copy(cnt_v, cnt_hbm.at[c, s])


def _tc_lin_r(x_ref, w_ref, b_ref, o_ref):
    o_ref[...] = (
        jnp.dot(x_ref[...], w_ref[...], preferred_element_type=jnp.float32)
        + b_ref[...]
    )


def _tc_combine(s_ref, c_ref, xr_ref, w_ref, o_ref):
    ones16 = jnp.ones((16, 1), jnp.float32)
    cnt_col = lax.dot_general(
        c_ref[...], ones16, (((0,), (0,)), ((), ())),
        preferred_element_type=jnp.float32,
    )
    mean = s_ref[...] / jnp.clip(cnt_col, 1.0, None)
    o_ref[...] = (
        jnp.dot(mean, w_ref[...], preferred_element_type=jnp.float32)
        + xr_ref[...]
    )


def kernel(x, edge_index, W_l, b_l, W_r):
    ei = edge_index.astype(jnp.int32)
    src = ei[0].reshape(NCHUNKS, CHUNK)
    dst = ei[1].reshape(NCHUNKS, CHUNK)

    xa = jnp.zeros((NP, D), dtype=jnp.float32).at[:N, :].set(x)

    sum_p, cnt_p = _sc_segment_sum(xa, src, dst)
    summed = sum_p.reshape(NP, D)
    cnt16 = cnt_p.transpose(1, 0, 2).reshape(16, NP)

    xr = pl.pallas_call(
        _tc_lin_r,
        grid=(NP // RB,),
        in_specs=[
            pl.BlockSpec((RB, D), lambda i: (i, 0)),
            pl.BlockSpec((D, D), lambda i: (0, 0)),
            pl.BlockSpec((1, D), lambda i: (0, 0)),
        ],
        out_specs=pl.BlockSpec((RB, D), lambda i: (i, 0)),
        out_shape=jax.ShapeDtypeStruct((NP, D), jnp.float32),
    )(xa, W_r.T, b_l.reshape(1, D))

    out = pl.pallas_call(
        _tc_combine,
        grid=(NP // RB,),
        in_specs=[
            pl.BlockSpec((RB, D), lambda i: (i, 0)),
            pl.BlockSpec((16, RB), lambda i: (0, i)),
            pl.BlockSpec((RB, D), lambda i: (i, 0)),
            pl.BlockSpec((D, D), lambda i: (0, 0)),
        ],
        out_specs=pl.BlockSpec((RB, D), lambda i: (i, 0)),
        out_shape=jax.ShapeDtypeStruct((NP, D), jnp.float32),
    )(summed, cnt16, xr, W_l.T)

    return out[:N]
